# baseline (speedup 1.0000x reference)
import functools

import jax
import jax.numpy as jnp
from jax import lax
from jax.experimental import pallas as pl
from jax.experimental.pallas import tpu as pltpu
from jax.experimental.pallas import tpu_sc as plsc

_VOCAB = 1000
_DMODEL = 128
_BATCH = 4096
_SEQ = 20

_NW = 32
_BB = _BATCH // _NW
_GRP = _BB // 16
_VC = 8
_NCHK = _VOCAB // _VC
_NOB = 4
_TOKW = _SEQ * _BB


def _matmul_body(w_ref, emb_ref, t_ref):
    t_ref[...] = lax.dot_general(
        w_ref[...], emb_ref[...],
        dimension_numbers=(((1,), (1,)), ((), ())),
        preferred_element_type=jnp.float32,
    )


def _fused_table_t(embedding, lm_head_w):
    return pl.pallas_call(
        _matmul_body,
        out_shape=jax.ShapeDtypeStruct((_VOCAB, _VOCAB), jnp.float32),
    )(lm_head_w, embedding)


def _gather_body(t_hbm, ids_hbm, out_hbm, idx_v, tch, obuf, tab_sh, tsem, osem):
    cid = lax.axis_index("c")
    sid = lax.axis_index("s")
    wid = sid * 2 + cid

    @pl.when(sid == 0)
    def _():
        pltpu.sync_copy(t_hbm, tab_sh)

    pltpu.sync_copy(ids_hbm.at[pl.ds(wid * _TOKW, _TOKW)], idx_v)
    plsc.subcore_barrier()

    b0 = wid * _BB
    _H = _VC * _VOCAB // 2

    def tch_cp(c, slot, h):
        return pltpu.make_async_copy(
            tab_sh.at[pl.ds(c * (_VC * _VOCAB) + h * _H, _H)],
            tch.at[pl.ds(slot * (_VC * _VOCAB) + h * _H, _H)],
            tsem.at[slot],
        )

    def slab_cp(c, l, li):
        return pltpu.make_async_copy(
            obuf.at[li],
            out_hbm.at[pl.ds(l * _VOCAB + c * _VC, _VC), pl.ds(b0, _BB)],
            osem.at[li],
        )

    tch_cp(0, 0, 0).start()
    tch_cp(0, 0, 1).start()

    def chunk_body(c, carry):
        slot = c % 2
        tch_cp(c, slot, 0).wait()
        tch_cp(c, slot, 1).wait()

        @pl.when(c < _NCHK - 1)
        def _():
            tch_cp(c + 1, 1 - slot, 0).start()
            tch_cp(c + 1, 1 - slot, 1).start()

        sbase = slot * (_VC * _VOCAB)

        def l_block(lb, carry2):
            for li in range(_NOB):
                l = lb * _NOB + li

                @pl.when(jnp.logical_or(c > 0, lb > 0))
                def _():
                    slab_cp(c, l, li).wait()

                for g in range(_GRP):
                    idxv = idx_v[pl.ds(l * _BB + g * 16, 16)] + sbase

                    @plsc.parallel_loop(0, _VC, step=1, unroll=8)
                    def _(vv):
                        obuf[li, vv, pl.ds(g * 16, 16)] = plsc.load_gather(
                            tch, [idxv + vv * _VOCAB]
                        )
                slab_cp(c, l, li).start()
            return carry2

        lax.fori_loop(0, _SEQ // _NOB, l_block, 0, unroll=False)
        return carry

    lax.fori_loop(0, _NCHK, chunk_body, 0, unroll=False)

    for li in range(_NOB):
        slab_cp(_NCHK - 1, (_SEQ // _NOB - 1) * _NOB + li, li).wait()


@functools.partial(
    pl.kernel,
    out_type=jax.ShapeDtypeStruct((_SEQ * _VOCAB, _BATCH), jnp.float32),
    mesh=plsc.VectorSubcoreMesh(core_axis_name="c", subcore_axis_name="s"),
    scratch_types=[
        pltpu.VMEM((_TOKW,), jnp.int32),
        pltpu.VMEM((2 * _VC * _VOCAB,), jnp.float32),
        pltpu.VMEM((_NOB, _VC, _BB), jnp.float32),
        pltpu.VMEM_SHARED((_VOCAB * _VOCAB,), jnp.float32),
        pltpu.SemaphoreType.DMA((2,)),
        pltpu.SemaphoreType.DMA((_NOB,)),
    ],
    compiler_params=pltpu.CompilerParams(needs_layout_passes=False),
)
def _gather_call(t_hbm, ids_hbm, out_hbm, idx_v, tch, obuf, tab_sh, tsem, osem):
    _gather_body(t_hbm, ids_hbm, out_hbm, idx_v, tch, obuf, tab_sh, tsem, osem)


def kernel(input_ids, embedding, lm_head_w):
    t = _fused_table_t(embedding, lm_head_w).reshape(_VOCAB * _VOCAB)
    ids = (
        input_ids.astype(jnp.int32)
        .reshape(_NW, _BB, _SEQ)
        .transpose(0, 2, 1)
        .reshape(_NW * _TOKW)
    )
    out = _gather_call(t, ids)
    return out.reshape(_SEQ, _VOCAB, _BATCH).transpose(2, 0, 1)


# speedup vs baseline: 1.4849x; 1.4849x over previous
import functools

import jax
import jax.numpy as jnp
from jax import lax
from jax.experimental import pallas as pl
from jax.experimental.pallas import tpu as pltpu
from jax.experimental.pallas import tpu_sc as plsc

_VOCAB = 1000
_DMODEL = 128
_BATCH = 4096
_SEQ = 20

_NW = 32
_BB = _BATCH // _NW
_GRP = _BB // 16
_VC = 40
_NCHK = _VOCAB // _VC
_NOB = 4
_TOKW = _SEQ * _BB


def _matmul_body(w_ref, emb_ref, t_ref):
    t_ref[...] = lax.dot_general(
        w_ref[...], emb_ref[...],
        dimension_numbers=(((1,), (1,)), ((), ())),
        preferred_element_type=jnp.float32,
    )


def _fused_table_t(embedding, lm_head_w):
    return pl.pallas_call(
        _matmul_body,
        out_shape=jax.ShapeDtypeStruct((_VOCAB, _VOCAB), jnp.float32),
    )(lm_head_w, embedding)


def _gather_body(t_hbm, ids_hbm, out_hbm, idx_v, tch, obuf, tsem, osem):
    cid = lax.axis_index("c")
    sid = lax.axis_index("s")
    wid = sid * 2 + cid

    pltpu.sync_copy(ids_hbm.at[pl.ds(wid * _TOKW, _TOKW)], idx_v)

    b0 = wid * _BB

    def tch_cp(c, slot):
        return pltpu.make_async_copy(
            t_hbm.at[pl.ds(c * (_VC * _VOCAB), _VC * _VOCAB)],
            tch.at[pl.ds(slot * (_VC * _VOCAB), _VC * _VOCAB)],
            tsem.at[slot],
        )

    def slab_cp(c, l, li):
        return pltpu.make_async_copy(
            obuf.at[li],
            out_hbm.at[pl.ds(l * _VOCAB + c * _VC, _VC), pl.ds(b0, _BB)],
            osem.at[li],
        )

    tch_cp(0, 0).start()

    def chunk_body(c, carry):
        slot = c % 2
        tch_cp(c, slot).wait()

        @pl.when(c < _NCHK - 1)
        def _():
            tch_cp(c + 1, 1 - slot).start()

        sbase = slot * (_VC * _VOCAB)

        def l_block(lb, carry2):
            for li in range(_NOB):
                l = lb * _NOB + li

                @pl.when(jnp.logical_or(c > 0, lb > 0))
                def _():
                    slab_cp(c, l, li).wait()

                for g in range(_GRP):
                    idxv = idx_v[pl.ds(l * _BB + g * 16, 16)] + sbase

                    @plsc.parallel_loop(0, _VC, step=1, unroll=8)
                    def _(vv):
                        obuf[li, vv, pl.ds(g * 16, 16)] = plsc.load_gather(
                            tch, [idxv + vv * _VOCAB]
                        )
                slab_cp(c, l, li).start()
            return carry2

        lax.fori_loop(0, _SEQ // _NOB, l_block, 0, unroll=False)
        return carry

    lax.fori_loop(0, _NCHK, chunk_body, 0, unroll=False)

    for li in range(_NOB):
        slab_cp(_NCHK - 1, (_SEQ // _NOB - 1) * _NOB + li, li).wait()


@functools.partial(
    pl.kernel,
    out_type=jax.ShapeDtypeStruct((_SEQ * _VOCAB, _BATCH), jnp.float32),
    mesh=plsc.VectorSubcoreMesh(core_axis_name="c", subcore_axis_name="s"),
    scratch_types=[
        pltpu.VMEM((_TOKW,), jnp.int32),
        pltpu.VMEM((2 * _VC * _VOCAB,), jnp.float32),
        pltpu.VMEM((_NOB, _VC, _BB), jnp.float32),
        pltpu.SemaphoreType.DMA((2,)),
        pltpu.SemaphoreType.DMA((_NOB,)),
    ],
    compiler_params=pltpu.CompilerParams(needs_layout_passes=False),
)
def _gather_call(t_hbm, ids_hbm, out_hbm, idx_v, tch, obuf, tsem, osem):
    _gather_body(t_hbm, ids_hbm, out_hbm, idx_v, tch, obuf, tsem, osem)


def kernel(input_ids, embedding, lm_head_w):
    t = _fused_table_t(embedding, lm_head_w).reshape(_VOCAB * _VOCAB)
    ids = (
        input_ids.astype(jnp.int32)
        .reshape(_NW, _BB, _SEQ)
        .transpose(0, 2, 1)
        .reshape(_NW * _TOKW)
    )
    out = _gather_call(t, ids)
    return out.reshape(_SEQ, _VOCAB, _BATCH).transpose(2, 0, 1)
